# baseline (speedup 1.0000x reference)
import functools

import jax
import jax.numpy as jnp
from jax import lax
from jax.experimental import pallas as pl
from jax.experimental.pallas import tpu as pltpu
from jax.experimental.pallas import tpu_sc as plsc

_MASK_PERCENT = 0.5
_NBUF = 4


def _mask_body(r_ref, m_ref):
    freq, cb = r_ref.shape
    keep_thresh = float(int(_MASK_PERCENT * freq))
    bits = lax.bitcast_convert_type(r_ref[...], jnp.int32)
    fidx = lax.broadcasted_iota(jnp.int32, (freq, cb), 0)
    key = (bits & jnp.int32(~63)) | fidx
    less = key[None, :, :] < key[:, None, :]
    rank = jnp.sum(less.astype(jnp.float32), axis=1)
    m_ref[...] = (rank >= keep_thresh).astype(jnp.float32)


def _compute_mask(r2):
    freq, ncols = r2.shape
    cb = 512
    return pl.pallas_call(
        _mask_body,
        grid=(ncols // cb,),
        in_specs=[pl.BlockSpec((freq, cb), lambda g: (0, g))],
        out_specs=pl.BlockSpec((freq, cb), lambda g: (0, g)),
        out_shape=jax.ShapeDtypeStruct((freq, ncols), jnp.float32),
    )(r2)


def _sc_mul_body(m_hbm, x_hbm, o_hbm, m_v, xb_v, ob_v, in_sem, out_sem):
    freq = x_hbm.shape[0]
    length = x_hbm.shape[2]
    n2 = x_hbm.shape[3]
    nlanes = 16
    cid = lax.axis_index("c")
    sid = lax.axis_index("s")
    wid = sid * 2 + cid
    g_per_w = x_hbm.shape[1] // 32
    gbase = wid * g_per_w
    slots = 2 * freq

    def start_in(i, b, p):
        f = i // 2
        g = gbase + 2 * p + (i % 2)
        pltpu.async_copy(x_hbm.at[f, g], xb_v.at[b], in_sem.at[b])

    def wait_in(b):
        pltpu.make_async_copy(x_hbm.at[0, 0], xb_v.at[b], in_sem.at[b]).wait()

    def start_out(i, b, p):
        f = i // 2
        g = gbase + 2 * p + (i % 2)
        pltpu.async_copy(ob_v.at[b], o_hbm.at[f, g], out_sem.at[b])

    def wait_out(b):
        pltpu.make_async_copy(ob_v.at[b], o_hbm.at[0, 0], out_sem.at[b]).wait()

    def compute(i, b):
        f = i // 2
        off = (i % 2) * n2
        for k in range(n2 // nlanes):
            mk = m_v[f, pl.ds(off + k * nlanes, nlanes)]
            for l in range(length):
                xv = xb_v[b, l, pl.ds(k * nlanes, nlanes)]
                ob_v[b, l, pl.ds(k * nlanes, nlanes)] = xv * mk

    for p in range(g_per_w // 2):
        pltpu.sync_copy(m_hbm.at[:, pl.ds((gbase + 2 * p) * n2, 2 * n2)], m_v)
        for b in range(_NBUF):
            start_in(b, b, p)

        def slot(i, b, p):
            @pl.when(i >= _NBUF)
            def _():
                wait_out(b)
            wait_in(b)
            compute(i, b)
            start_out(i, b, p)
            @pl.when(i + _NBUF < slots)
            def _():
                start_in(i + _NBUF, b, p)

        def chunk(c, carry, p=p):
            i0 = c * _NBUF
            for b in range(_NBUF):
                slot(i0 + b, b, p)
            return carry

        lax.fori_loop(0, slots // _NBUF, chunk, 0)
        for b in range(_NBUF):
            wait_out(b)


def kernel(x):
    freq, batch, n1, n2, length = x.shape
    ncols = batch * n1 * n2
    rkey = jax.random.key(42)
    r2 = jax.random.uniform(rkey, (freq, ncols), dtype=jnp.float32)
    m2 = _compute_mask(r2)

    xt = jnp.transpose(x, (0, 1, 2, 4, 3))
    x4 = xt.reshape(freq, batch * n1, length, n2)

    mesh = plsc.VectorSubcoreMesh(core_axis_name="c", subcore_axis_name="s")
    sc_mul = functools.partial(
        pl.kernel,
        out_type=jax.ShapeDtypeStruct((freq, batch * n1, length, n2),
                                      jnp.float32),
        mesh=mesh,
        scratch_types=[
            pltpu.VMEM((freq, 2 * n2), jnp.float32),
            pltpu.VMEM((_NBUF, length, n2), jnp.float32),
            pltpu.VMEM((_NBUF, length, n2), jnp.float32),
            pltpu.SemaphoreType.DMA((_NBUF,)),
            pltpu.SemaphoreType.DMA((_NBUF,)),
        ],
    )(_sc_mul_body)
    out = sc_mul(m2, x4)
    out5 = out.reshape(freq, batch, n1, length, n2)
    return jnp.transpose(out5, (0, 1, 2, 4, 3))


# speedup vs baseline: 1.2553x; 1.2553x over previous
import functools

import jax
import jax.numpy as jnp
from jax import lax
from jax.experimental import pallas as pl
from jax.experimental.pallas import tpu as pltpu
from jax.experimental.pallas import tpu_sc as plsc

_MASK_PERCENT = 0.5
_NBUF = 2


def _mask_body(r_ref, m_ref):
    freq, cb = r_ref.shape
    keep_thresh = float(int(_MASK_PERCENT * freq))
    bits = lax.bitcast_convert_type(r_ref[...], jnp.int32)
    fidx = lax.broadcasted_iota(jnp.int32, (freq, cb), 0)
    key = (bits & jnp.int32(~63)) | fidx
    less = key[None, :, :] < key[:, None, :]
    rank = jnp.sum(less.astype(jnp.float32), axis=1)
    m_ref[...] = (rank >= keep_thresh).astype(jnp.float32)


def _compute_mask(r2):
    freq, ncols = r2.shape
    cb = 512
    return pl.pallas_call(
        _mask_body,
        grid=(ncols // cb,),
        in_specs=[pl.BlockSpec((freq, cb), lambda g: (0, g))],
        out_specs=pl.BlockSpec((freq, cb), lambda g: (0, g)),
        out_shape=jax.ShapeDtypeStruct((freq, ncols), jnp.float32),
    )(r2)


def _sc_mul_body(m_hbm, x_hbm, o_hbm, m_v, xb_v, ob_v, in_sem, out_sem):
    freq = x_hbm.shape[0]
    length = x_hbm.shape[2]
    n2 = x_hbm.shape[3]
    nlanes = 16
    cid = lax.axis_index("c")
    sid = lax.axis_index("s")
    wid = sid * 2 + cid
    g_per_w = x_hbm.shape[1] // 32
    gbase = wid * g_per_w
    slots = freq

    def start_in(f, b):
        pltpu.async_copy(x_hbm.at[f, pl.ds(gbase, g_per_w)], xb_v.at[b],
                         in_sem.at[b])

    def wait_in(b):
        pltpu.make_async_copy(x_hbm.at[0, pl.ds(0, g_per_w)], xb_v.at[b],
                              in_sem.at[b]).wait()

    def start_out(f, b):
        pltpu.async_copy(ob_v.at[b], o_hbm.at[f, pl.ds(gbase, g_per_w)],
                         out_sem.at[b])

    def wait_out(b):
        pltpu.make_async_copy(ob_v.at[b], o_hbm.at[0, pl.ds(0, g_per_w)],
                              out_sem.at[b]).wait()

    def compute(f, b):
        for gl in range(g_per_w):
            for k in range(n2 // nlanes):
                mk = m_v[f, pl.ds(gl * n2 + k * nlanes, nlanes)]
                for l in range(length):
                    xv = xb_v[b, gl, l, pl.ds(k * nlanes, nlanes)]
                    ob_v[b, gl, l, pl.ds(k * nlanes, nlanes)] = xv * mk

    pltpu.sync_copy(m_hbm.at[:, pl.ds(gbase * n2, g_per_w * n2)], m_v)
    for b in range(_NBUF):
        start_in(b, b)

    def slot(f, b):
        @pl.when(f >= _NBUF)
        def _():
            wait_out(b)
        wait_in(b)
        compute(f, b)
        start_out(f, b)
        @pl.when(f + _NBUF < slots)
        def _():
            start_in(f + _NBUF, b)

    def chunk(c, carry):
        f0 = c * _NBUF
        for b in range(_NBUF):
            slot(f0 + b, b)
        return carry

    lax.fori_loop(0, slots // _NBUF, chunk, 0)
    for b in range(_NBUF):
        wait_out(b)


def kernel(x):
    freq, batch, n1, n2, length = x.shape
    ncols = batch * n1 * n2
    rkey = jax.random.key(42)
    r2 = jax.random.uniform(rkey, (freq, ncols), dtype=jnp.float32)
    m2 = _compute_mask(r2)

    xt = jnp.transpose(x, (0, 1, 2, 4, 3))
    x4 = xt.reshape(freq, batch * n1, length, n2)

    mesh = plsc.VectorSubcoreMesh(core_axis_name="c", subcore_axis_name="s")
    g_per_w = (batch * n1) // 32
    sc_mul = functools.partial(
        pl.kernel,
        out_type=jax.ShapeDtypeStruct((freq, batch * n1, length, n2),
                                      jnp.float32),
        mesh=mesh,
        scratch_types=[
            pltpu.VMEM((freq, g_per_w * n2), jnp.float32),
            pltpu.VMEM((_NBUF, g_per_w, length, n2), jnp.float32),
            pltpu.VMEM((_NBUF, g_per_w, length, n2), jnp.float32),
            pltpu.SemaphoreType.DMA((_NBUF,)),
            pltpu.SemaphoreType.DMA((_NBUF,)),
        ],
    )(_sc_mul_body)
    out = sc_mul(m2, x4)
    out5 = out.reshape(freq, batch, n1, length, n2)
    return jnp.transpose(out5, (0, 1, 2, 4, 3))
